# baseline (speedup 1.0000x reference)
import jax
import jax.numpy as jnp
from jax import lax
from jax.experimental import pallas as pl
from jax.experimental.pallas import tpu as pltpu
from jax.experimental.pallas import tpu_sc as plsc

C, G, S, L, B = 4, 8192, 8, 4, 16
INFER_STEP = 2
GAMMA = 0.01
INVG = float(1.0 / GAMMA)

NC, NS = 2, 16
NW = NC * NS
BH = B // 2
NGQ = 4
GQ = G // NGQ
NG = 16
SL = S * L
CHUNK_IDX = NG * SL
NCHUNK = GQ // NG
TBL = BH * G


def _make_sc(table_has_clause_dim):
    def body(tab, idx_hbm, m_out, s_out, table_v, idx_v, pbuf, mslab, sslab,
             tsem, isem0, isem1):
        wid = lax.axis_index("s") * NC + lax.axis_index("c")
        c = wid // 8
        rem = wid - c * 8
        hb = rem // NGQ
        qg = rem - hb * NGQ
        b0 = hb * BH
        g0 = qg * GQ

        tcps = []
        for j in range(BH):
            if table_has_clause_dim:
                src = tab.at[c, b0 + j]
            else:
                src = tab.at[b0 + j]
            tcps.append(
                pltpu.async_copy(src, table_v.at[pl.ds(j * G, G)], tsem))

        isems = (isem0, isem1)

        def idx_src(k):
            kk = jnp.minimum(k, NCHUNK - 1)
            return idx_hbm.at[c, pl.ds(g0 + kk * NG, NG)]

        def start_idx(k, p):
            pltpu.async_copy(idx_src(k), idx_v.at[p], isems[p])

        def wait_idx(p):
            pltpu.make_async_copy(idx_src(0), idx_v.at[p], isems[p]).wait()

        start_idx(0, 0)
        start_idx(1, 1)
        for cp in tcps:
            cp.wait()

        iota16 = lax.iota(jnp.int32, 16)

        def compute(k, p):
            goff = k * NG
            for s in range(S):
                svec = jnp.full((16,), s, jnp.int32)
                prods = [None] * BH
                for l in range(L):
                    lvec = jnp.full((16,), l, jnp.int32)
                    gidx = plsc.load_gather(idx_v.at[p], [iota16, svec, lvec])
                    for j in range(BH):
                        fidx = gidx + (j * G) if j > 0 else gidx
                        v = plsc.load_gather(table_v, [fidx])
                        if l == 0:
                            prods[j] = v
                        else:
                            prods[j] = prods[j] * v
                for j in range(BH):
                    pbuf[s, j] = prods[j]
            for j in range(BH):
                ps = [pbuf[s, j] for s in range(S)]
                m = ps[0]
                for s in range(1, S):
                    m = jnp.maximum(m, ps[s])
                acc = jnp.exp((ps[0] - m) * INVG)
                for s in range(1, S):
                    acc = acc + jnp.exp((ps[s] - m) * INVG)
                mslab[j, pl.ds(goff, NG)] = m
                sslab[j, pl.ds(goff, NG)] = acc

        def outer(i, carry):
            for u in range(2):
                k = i * 2 + u
                wait_idx(u)
                compute(k, u)
                start_idx(k + 2, u)
            return carry

        lax.fori_loop(0, NCHUNK // 2, outer, 0)
        wait_idx(0)
        wait_idx(1)

        pltpu.sync_copy(mslab, m_out.at[c, pl.ds(b0, BH), pl.ds(g0, GQ)])
        pltpu.sync_copy(sslab, s_out.at[c, pl.ds(b0, BH), pl.ds(g0, GQ)])

    tab_shape = (C, B, G) if table_has_clause_dim else (B, G)
    return pl.kernel(
        body,
        out_type=(
            jax.ShapeDtypeStruct((C, B, G), jnp.float32),
            jax.ShapeDtypeStruct((C, B, G), jnp.float32),
        ),
        mesh=plsc.VectorSubcoreMesh(
            core_axis_name="c", subcore_axis_name="s",
            num_cores=NC, num_subcores=NS,
        ),
        scratch_types=[
            pltpu.VMEM((TBL,), jnp.float32),
            pltpu.VMEM((2, NG, S, L), jnp.int32),
            pltpu.VMEM((S, BH, NG), jnp.float32),
            pltpu.VMEM((BH, GQ), jnp.float32),
            pltpu.VMEM((BH, GQ), jnp.float32),
            pltpu.SemaphoreType.DMA,
            pltpu.SemaphoreType.DMA,
            pltpu.SemaphoreType.DMA,
        ],
        compiler_params=pltpu.CompilerParams(
            use_tc_tiling_on_sc=False, needs_layout_passes=False
        ),
    )


_sc_step1 = _make_sc(False)
_sc_step2 = _make_sc(True)


def _tc_body(R_ref, m_ref, s_ref, out_ref):
    t = m_ref[:] + GAMMA * jnp.log(s_ref[:])
    mx = jnp.max(t, axis=1, keepdims=True)
    r = t / jnp.maximum(mx, 1.0)
    Rc = R_ref[:]
    mm = jnp.maximum(Rc, r)
    u = mm + GAMMA * jnp.log(
        jnp.exp((Rc - mm) * INVG) + jnp.exp((r - mm) * INVG)
    )
    M = jnp.max(u)
    out_ref[:] = u / jnp.maximum(M, 1.0)


_tc_combine = pl.pallas_call(
    _tc_body,
    out_shape=jax.ShapeDtypeStruct((C, B * G), jnp.float32),
)


def kernel(x, I):
    Rflat = jnp.broadcast_to(x.reshape(1, B * G), (C, B * G))
    m, acc = _sc_step1(x, I)
    Rflat = _tc_combine(Rflat, m.reshape(C, B * G), acc.reshape(C, B * G))
    for _ in range(INFER_STEP - 1):
        m, acc = _sc_step2(Rflat.reshape(C, B, G), I)
        Rflat = _tc_combine(Rflat, m.reshape(C, B * G), acc.reshape(C, B * G))
    return Rflat.reshape(C, B, G)


# speedup vs baseline: 1.2477x; 1.2477x over previous
import jax
import jax.numpy as jnp
from jax import lax
from jax.experimental import pallas as pl
from jax.experimental.pallas import tpu as pltpu
from jax.experimental.pallas import tpu_sc as plsc

C, G, S, L, B = 4, 8192, 8, 4, 16
INFER_STEP = 2
GAMMA = 0.01
INVG = float(1.0 / GAMMA)

NC, NS = 2, 16
NW = NC * NS
NG = 16
RPC = NG * S * L
IDX_ROWS = RPC // 128
NCHUNK = G // (NW * NG)
RING = 4


def _sc_body(tab, idx_hbm, m_out, s_out, idx_v, rows_v, mbuf, sbuf, *sems):
    rows_sems = sems[0:4]
    idx_sems = sems[4:8]
    out_sems = sems[8:12]

    wid = lax.axis_index("s") * NC + lax.axis_index("c")
    base = wid * NCHUNK

    def idx_slice(off):
        q = base + jnp.minimum(off, NCHUNK - 1)
        return idx_hbm.at[pl.ds(q * IDX_ROWS, IDX_ROWS)]

    def start_idx(off, p):
        pltpu.async_copy(idx_slice(off), idx_v.at[p], idx_sems[p])

    def wait_idx(p):
        pltpu.make_async_copy(idx_slice(0), idx_v.at[p], idx_sems[p]).wait()

    def start_gathers(p):
        for j in range(IDX_ROWS):
            pltpu.async_copy(tab.at[idx_v.at[p, j]], rows_v.at[p, j],
                             rows_sems[p])

    def wait_gathers(p):
        for j in range(IDX_ROWS):
            pltpu.make_async_copy(tab.at[idx_v.at[p, j]], rows_v.at[p, j],
                                  rows_sems[p]).wait()

    def out_slices(off):
        q = base + off
        return (m_out.at[pl.ds(q * NG * B, NG * B)],
                s_out.at[pl.ds(q * NG * B, NG * B)])

    def start_out(off, p):
        mo, so = out_slices(off)
        pltpu.async_copy(mbuf.at[p], mo, out_sems[p])
        pltpu.async_copy(sbuf.at[p], so, out_sems[p])

    def wait_out(p):
        mo, so = out_slices(0)
        pltpu.make_async_copy(mbuf.at[p], mo, out_sems[p]).wait()
        pltpu.make_async_copy(sbuf.at[p], so, out_sems[p]).wait()

    def compute(p):
        for gl in range(NG):
            r0 = gl * S * L
            ps = []
            for s in range(S):
                k = r0 + s * L
                v = rows_v[p, k // 128, k % 128]
                for l in range(1, L):
                    v = v * rows_v[p, (k + l) // 128, (k + l) % 128]
                ps.append(v)
            m = ps[0]
            for s in range(1, S):
                m = jnp.maximum(m, ps[s])
            acc = jnp.exp((ps[0] - m) * INVG)
            for s in range(1, S):
                acc = acc + jnp.exp((ps[s] - m) * INVG)
            mbuf[p, pl.ds(gl * B, B)] = m
            sbuf[p, pl.ds(gl * B, B)] = acc

    pltpu.sync_copy(idx_slice(0), idx_v.at[0])
    start_gathers(0)
    start_idx(1, 1)
    wait_idx(1)
    start_gathers(1)
    start_idx(2, 2)

    def outer(i, carry):
        off0 = i * RING
        for u in range(RING):
            off = off0 + u
            p = u
            p2 = (u + 2) % RING
            p3 = (u + 3) % RING
            wait_idx(p2)
            start_gathers(p2)
            wait_gathers(p)
            start_idx(off + 3, p3)
            @pl.when(off >= RING)
            def _():
                wait_out(p)
            compute(p)
            start_out(off, p)
        return carry

    lax.fori_loop(0, NCHUNK // RING, outer, 0)

    wait_gathers(0)
    wait_gathers(1)
    wait_idx(2)
    for p in range(RING):
        wait_out(p)


_sc_clause = pl.kernel(
    _sc_body,
    out_type=(
        jax.ShapeDtypeStruct((G * B,), jnp.float32),
        jax.ShapeDtypeStruct((G * B,), jnp.float32),
    ),
    mesh=plsc.VectorSubcoreMesh(
        core_axis_name="c", subcore_axis_name="s", num_cores=NC, num_subcores=NS
    ),
    scratch_types=[
        pltpu.VMEM((RING, IDX_ROWS, 128), jnp.int32),
        pltpu.VMEM((RING, IDX_ROWS, 128, B), jnp.float32),
        pltpu.VMEM((RING, NG * B), jnp.float32),
        pltpu.VMEM((RING, NG * B), jnp.float32),
    ] + [pltpu.SemaphoreType.DMA] * 12,
    compiler_params=pltpu.CompilerParams(use_tc_tiling_on_sc=False),
)


def _make_tc_combine(r_has_clause_dim):
    def body(R_ref, *refs):
        ms = refs[0:C]
        ss = refs[C:2 * C]
        out_ref = refs[2 * C]
        us = []
        M = None
        for c in range(C):
            t = ms[c][:] + GAMMA * jnp.log(ss[c][:])
            mx = jnp.max(t)
            r = t / jnp.maximum(mx, 1.0)
            Rc = R_ref[c] if r_has_clause_dim else R_ref[:]
            mm = jnp.maximum(Rc, r)
            u = mm + GAMMA * jnp.log(
                jnp.exp((Rc - mm) * INVG) + jnp.exp((r - mm) * INVG)
            )
            us.append(u)
            uM = jnp.max(u)
            M = uM if M is None else jnp.maximum(M, uM)
        scale = 1.0 / jnp.maximum(M, 1.0)
        for c in range(C):
            out_ref[c] = us[c] * scale

    return pl.pallas_call(
        body,
        out_shape=jax.ShapeDtypeStruct((C, G * B), jnp.float32),
    )


_tc_combine1 = _make_tc_combine(False)
_tc_combine2 = _make_tc_combine(True)


def kernel(x, I):
    xT = x.T
    parts = [_sc_clause(xT, I[c].reshape(G * S * L // 128, 128))
             for c in range(C)]
    ms = [p[0] for p in parts]
    ss = [p[1] for p in parts]
    Rt = _tc_combine1(xT.reshape(G * B), *ms, *ss)
    for _ in range(INFER_STEP - 1):
        parts = [_sc_clause(Rt[c].reshape(G, B),
                            I[c].reshape(G * S * L // 128, 128))
                 for c in range(C)]
        ms = [p[0] for p in parts]
        ss = [p[1] for p in parts]
        Rt = _tc_combine2(Rt, *ms, *ss)
    return Rt.reshape(C, G, B).transpose(0, 2, 1)
